# baseline (speedup 1.0000x reference)
import jax
import jax.numpy as jnp
from jax.experimental import pallas as pl

B, R, L, IN, OUT = 128, 6, 128, 128, 64
NB = 4


def _aergcn_body(text_ref, adj_ref, w2_ref, swd_ref, sb_ref, out_ref):
    w2 = w2_ref[...]
    swd = swd_ref[...]
    sb = sb_ref[0, 0]
    ones_col = jnp.ones((L, 1), jnp.bfloat16)
    for b in range(NB):
        x = text_ref[b].astype(jnp.bfloat16)
        hid = jnp.dot(x, w2, preferred_element_type=jnp.float32)
        hid_bf = hid.astype(jnp.bfloat16)
        sv = jnp.dot(hid_bf, swd, preferred_element_type=jnp.float32)
        sv_bf = sv.astype(jnp.bfloat16)
        acc = jnp.zeros((L, OUT), jnp.float32)
        zsum = jnp.zeros((L, 1), jnp.float32)
        mrun = jnp.full((L, 1), -1e30, jnp.float32)
        for r in range(R):
            a = adj_ref[b, r].astype(jnp.bfloat16)
            rhs = jnp.concatenate(
                [hid_bf[:, r * OUT:(r + 1) * OUT], sv_bf[:, r:r + 1], ones_col],
                axis=1)
            md = jnp.dot(a, rhs, preferred_element_type=jnp.float32)
            msg = md[:, :OUT]
            snum = md[:, OUT:OUT + 1]
            den = md[:, OUT + 1:OUT + 2]
            recip = jnp.where(den == 0.0, 1.0, 1.0 / den)
            div = msg * recip
            s = snum * recip + sb
            mnew = jnp.maximum(mrun, s)
            corr = jnp.exp(mrun - mnew)
            e = jnp.exp(s - mnew)
            zsum = zsum * corr + e
            acc = acc * corr + e * div
            mrun = mnew
        out_ref[b] = acc / zsum


@jax.jit
def kernel(text, adj, weight, score_w, score_b):
    w2 = weight.transpose(1, 0, 2).reshape(IN, R * OUT).astype(jnp.bfloat16)
    eye = jnp.eye(R, dtype=jnp.float32)
    swd = (eye[:, None, :] * score_w[0][None, :, None]).reshape(
        R * OUT, R).astype(jnp.bfloat16)
    sb = score_b.reshape(1, 1)
    grid = (B // NB,)
    return pl.pallas_call(
        _aergcn_body,
        grid=grid,
        in_specs=[
            pl.BlockSpec((NB, L, IN), lambda i: (i, 0, 0)),
            pl.BlockSpec((NB, R, L, L), lambda i: (i, 0, 0, 0)),
            pl.BlockSpec((IN, R * OUT), lambda i: (0, 0)),
            pl.BlockSpec((R * OUT, R), lambda i: (0, 0)),
            pl.BlockSpec((1, 1), lambda i: (0, 0)),
        ],
        out_specs=pl.BlockSpec((NB, L, OUT), lambda i: (i, 0, 0)),
        out_shape=jax.ShapeDtypeStruct((B, L, OUT), jnp.float32),
    )(text, adj, w2, swd, sb)


# speedup vs baseline: 2.6016x; 2.6016x over previous
import jax
import jax.numpy as jnp
from jax.experimental import pallas as pl

B, R, L, IN, OUT = 128, 6, 128, 128, 64
NB = 4


def _aergcn_body(text_ref, adj_ref, w2_ref, swd_ref, sb_ref, out_ref):
    w2 = w2_ref[...]
    swd = swd_ref[...]
    sb = sb_ref[0, 0]
    ones_row = jnp.ones((1, L), jnp.bfloat16)
    for b in range(NB):
        x = text_ref[b].astype(jnp.bfloat16)
        hT = jax.lax.dot_general(
            w2, x, (((0,), (1,)), ((), ())),
            preferred_element_type=jnp.float32).astype(jnp.bfloat16)
        svT = jnp.dot(swd, hT, preferred_element_type=jnp.float32
                      ).astype(jnp.bfloat16)
        accT = jnp.zeros((OUT, L), jnp.float32)
        zsum = jnp.zeros((1, L), jnp.float32)
        mrun = jnp.full((1, L), -1e30, jnp.float32)
        for r in range(R):
            a = adj_ref[b, r].astype(jnp.bfloat16)
            lhs = jnp.concatenate(
                [hT[r * OUT:(r + 1) * OUT], svT[r:r + 1], ones_row],
                axis=0)
            mdT = jax.lax.dot_general(
                lhs, a, (((1,), (1,)), ((), ())),
                preferred_element_type=jnp.float32)
            msgT = mdT[:OUT]
            snum = mdT[OUT:OUT + 1]
            den = mdT[OUT + 1:OUT + 2]
            recip = jnp.where(den == 0.0, 1.0, 1.0 / den)
            s = snum * recip + sb
            mnew = jnp.maximum(mrun, s)
            corr = jnp.exp(mrun - mnew)
            e = jnp.exp(s - mnew)
            zsum = zsum * corr + e
            accT = accT * corr + (e * recip) * msgT
            mrun = mnew
        outT = accT / zsum
        out_ref[b] = outT.T


@jax.jit
def kernel(text, adj, weight, score_w, score_b):
    w2 = weight.transpose(1, 0, 2).reshape(IN, R * OUT).astype(jnp.bfloat16)
    eye = jnp.eye(R, dtype=jnp.float32)
    swd = (eye[:, :, None] * score_w[0][None, None, :]).reshape(
        R, R * OUT).astype(jnp.bfloat16)
    sb = score_b.reshape(1, 1)
    grid = (B // NB,)
    return pl.pallas_call(
        _aergcn_body,
        grid=grid,
        in_specs=[
            pl.BlockSpec((NB, L, IN), lambda i: (i, 0, 0)),
            pl.BlockSpec((NB, R, L, L), lambda i: (i, 0, 0, 0)),
            pl.BlockSpec((IN, R * OUT), lambda i: (0, 0)),
            pl.BlockSpec((R, R * OUT), lambda i: (0, 0)),
            pl.BlockSpec((1, 1), lambda i: (0, 0)),
        ],
        out_specs=pl.BlockSpec((NB, L, OUT), lambda i: (i, 0, 0)),
        out_shape=jax.ShapeDtypeStruct((B, L, OUT), jnp.float32),
    )(text, adj, w2, swd, sb)


# speedup vs baseline: 3.2295x; 1.2413x over previous
import jax
import jax.numpy as jnp
from jax.experimental import pallas as pl

B, R, L, IN, OUT = 128, 6, 128, 128, 64
NB = 4
SUB = 72
KA = 136


def _aergcn_body(text_ref, adj_ref, waug_ref, out_ref):
    onescol = jnp.concatenate(
        [jnp.ones((L, 1), jnp.bfloat16), jnp.zeros((L, KA - IN - 1), jnp.bfloat16)],
        axis=1)
    for b in range(NB):
        x = jnp.concatenate(
            [text_ref[b].astype(jnp.bfloat16), onescol], axis=1)
        accT = jnp.zeros((OUT, L), jnp.float32)
        zsum = jnp.zeros((1, L), jnp.float32)
        for r in range(R):
            hT = jax.lax.dot_general(
                waug_ref[r], x, (((1,), (1,)), ((), ())),
                preferred_element_type=jnp.float32).astype(jnp.bfloat16)
            a = adj_ref[b, r].astype(jnp.bfloat16)
            mdT = jax.lax.dot_general(
                hT, a, (((1,), (1,)), ((), ())),
                preferred_element_type=jnp.float32)
            msgT = mdT[:OUT]
            snum = mdT[OUT:OUT + 1]
            den = mdT[OUT + 1:OUT + 2]
            recip = jnp.where(den == 0.0, 1.0, 1.0 / den)
            e = jnp.exp(snum * recip)
            zsum = zsum + e
            accT = accT + (e * recip) * msgT
        outT = accT / zsum
        out_ref[b] = outT.T


@jax.jit
def kernel(text, adj, weight, score_w, score_b):
    wT = weight.transpose(0, 2, 1)
    svT = jnp.einsum('rfo,o->rf', weight, score_w[0])[:, None, :]
    rows = jnp.concatenate(
        [wT, svT, jnp.zeros((R, SUB - OUT - 1, IN), jnp.float32)], axis=1)
    waug = jnp.concatenate(
        [rows, jnp.zeros((R, SUB, KA - IN), jnp.float32)], axis=2)
    waug = waug.at[:, OUT + 1, IN].set(1.0).astype(jnp.bfloat16)
    del score_b
    grid = (B // NB,)
    return pl.pallas_call(
        _aergcn_body,
        grid=grid,
        in_specs=[
            pl.BlockSpec((NB, L, IN), lambda i: (i, 0, 0)),
            pl.BlockSpec((NB, R, L, L), lambda i: (i, 0, 0, 0)),
            pl.BlockSpec((R, SUB, KA), lambda i: (0, 0, 0)),
        ],
        out_specs=pl.BlockSpec((NB, L, OUT), lambda i: (i, 0, 0)),
        out_shape=jax.ShapeDtypeStruct((B, L, OUT), jnp.float32),
    )(text, adj, waug)


# speedup vs baseline: 4.2814x; 1.3257x over previous
import jax
import jax.numpy as jnp
from jax.experimental import pallas as pl

B, R, L, IN, OUT = 128, 6, 128, 128, 64
NB = 16
SUB = 72
KA = 136


def _aergcn_body(text_ref, adj_ref, waug_ref, out_ref):
    onescol = jnp.concatenate(
        [jnp.ones((L, 1), jnp.bfloat16), jnp.zeros((L, KA - IN - 1), jnp.bfloat16)],
        axis=1)
    for b in range(NB):
        x = jnp.concatenate(
            [text_ref[b].astype(jnp.bfloat16), onescol], axis=1)
        accT = jnp.zeros((OUT, L), jnp.float32)
        zsum = jnp.zeros((1, L), jnp.float32)
        for r in range(R):
            hT = jax.lax.dot_general(
                waug_ref[r], x, (((1,), (1,)), ((), ())),
                preferred_element_type=jnp.float32).astype(jnp.bfloat16)
            a = adj_ref[b, r].astype(jnp.bfloat16)
            mdT = jax.lax.dot_general(
                hT, a, (((1,), (1,)), ((), ())),
                preferred_element_type=jnp.float32)
            msgT = mdT[:OUT]
            snum = mdT[OUT:OUT + 1]
            den = mdT[OUT + 1:OUT + 2]
            recip = jnp.where(den == 0.0, 1.0, 1.0 / den)
            e = jnp.exp(snum * recip)
            zsum = zsum + e
            accT = accT + (e * recip) * msgT
        outT = accT / zsum
        out_ref[b] = outT.T


@jax.jit
def kernel(text, adj, weight, score_w, score_b):
    wT = weight.transpose(0, 2, 1)
    svT = jnp.einsum('rfo,o->rf', weight, score_w[0])[:, None, :]
    rows = jnp.concatenate(
        [wT, svT, jnp.zeros((R, SUB - OUT - 1, IN), jnp.float32)], axis=1)
    waug = jnp.concatenate(
        [rows, jnp.zeros((R, SUB, KA - IN), jnp.float32)], axis=2)
    waug = waug.at[:, OUT + 1, IN].set(1.0).astype(jnp.bfloat16)
    del score_b
    grid = (B // NB,)
    return pl.pallas_call(
        _aergcn_body,
        grid=grid,
        in_specs=[
            pl.BlockSpec((NB, L, IN), lambda i: (i, 0, 0)),
            pl.BlockSpec((NB, R, L, L), lambda i: (i, 0, 0, 0)),
            pl.BlockSpec((R, SUB, KA), lambda i: (0, 0, 0)),
        ],
        out_specs=pl.BlockSpec((NB, L, OUT), lambda i: (i, 0, 0)),
        out_shape=jax.ShapeDtypeStruct((B, L, OUT), jnp.float32),
    )(text, adj, waug)
